# baseline (speedup 1.0000x reference)
import jax
import jax.numpy as jnp
from jax.experimental import pallas as pl
from jax.experimental.pallas import tpu as pltpu

N = 2048
FEAT = 128
HID = 32
HEADS = 8
CLASSES = 16
QV = 128
ALPHA = 0.2

BR = 256
NB = N // BR


def _han_kernel(x_ref, W_ref, a_ref, adj0_ref, adj1_ref, adj2_ref,
                Ws_ref, bs_ref, q_ref, Wo_ref, bo_ref, out_ref,
                hbt_s, ex_s, fx_s, f2m_s, z_s):
    i = pl.program_id(0)

    @pl.when(i == 0)
    def _prep():
        for hd in range(HEADS):
            ht = jax.lax.dot_general(
                W_ref[hd], x_ref[...], (((0,), (1,)), ((), ())),
                preferred_element_type=jnp.float32)
            a2 = a_ref[hd, HID:, :]
            f2f = jax.lax.dot_general(
                a2, ht, (((0,), (0,)), ((), ())),
                preferred_element_type=jnp.float32)
            f2max = jnp.max(f2f)
            f2m_s[hd, 0] = f2max
            ex_s[hd] = jnp.exp(f2f - f2max).astype(jnp.bfloat16)
            fx_s[hd] = jnp.exp(ALPHA * (f2f - f2max)).astype(jnp.bfloat16)
            hbt_s[hd] = jnp.concatenate(
                [ht, jnp.ones((1, N), jnp.float32)],
                axis=0).astype(jnp.bfloat16)

    adjs = (adj0_ref[...], adj1_ref[...], adj2_ref[...])
    for hd in range(HEADS):
        hbt = hbt_s[hd]
        hrows = hbt_s[hd, :HID, pl.ds(i * BR, BR)].astype(jnp.float32)
        f1b_f = jax.lax.dot_general(
            hrows, a_ref[hd, :HID, :], (((0,), (0,)), ((), ())),
            preferred_element_type=jnp.float32)
        mf = f1b_f + f2m_s[hd, 0]
        mb = jnp.where(mf >= 0, mf, ALPHA * mf)
        ci = jnp.exp(mf - mb).astype(jnp.bfloat16)
        di = jnp.exp(ALPHA * mf - mb).astype(jnp.bfloat16)
        pshared = jnp.maximum(ex_s[hd] * ci, fx_s[hd] * di)
        for path in range(3):
            p = pshared * adjs[path].astype(jnp.bfloat16)
            nd = jax.lax.dot_general(
                p, hbt, (((1,), (1,)), ((), ())),
                preferred_element_type=jnp.float32)
            denom = jnp.maximum(nd[:, HID:], 1e-38)
            att = nd[:, :HID] / denom
            z_s[hd, pl.ds(i * BR, BR), HID * path:HID * (path + 1)] = \
                jnp.where(att > 0, att,
                          jnp.exp(att) - 1.0).astype(jnp.bfloat16)

    @pl.when(i == NB - 1)
    def _semantic():
        wbars = []
        ys = []
        for path in range(3):
            t = jnp.zeros((N, QV), dtype=jnp.float32)
            y = jnp.zeros((N, CLASSES), dtype=jnp.float32)
            for h in range(HEADS):
                zh = z_s[h, :, HID * path:HID * (path + 1)]
                t = t + jnp.dot(zh, Ws_ref[h],
                                preferred_element_type=jnp.float32)
                y = y + jnp.dot(zh, Wo_ref[h],
                                preferred_element_type=jnp.float32)
            w = jnp.dot(jnp.tanh(t + bs_ref[...]), q_ref[...],
                        preferred_element_type=jnp.float32)
            wbars.append(jnp.sum(w) / N)
            ys.append(y)
        m = jnp.maximum(jnp.maximum(wbars[0], wbars[1]), wbars[2])
        e0 = jnp.exp(wbars[0] - m)
        e1 = jnp.exp(wbars[1] - m)
        e2 = jnp.exp(wbars[2] - m)
        den = e0 + e1 + e2
        out_ref[...] = (e0 * ys[0] + e1 * ys[1] + e2 * ys[2]) / den \
            + bo_ref[...]


def kernel(x, adj0, adj1, adj2, W, a, Ws, bs, q, Wo, bo):
    adj_spec = pl.BlockSpec((BR, N), lambda i: (i, 0))
    res2 = lambda i: (0, 0)
    res3 = lambda i: (0, 0, 0)
    return pl.pallas_call(
        _han_kernel,
        grid=(NB,),
        in_specs=[
            pl.BlockSpec((N, FEAT), res2),
            pl.BlockSpec((HEADS, FEAT, HID), res3),
            pl.BlockSpec((HEADS, 2 * HID, 1), res3),
            adj_spec, adj_spec, adj_spec,
            pl.BlockSpec((HEADS, HID, QV), res3),
            pl.BlockSpec((1, QV), res2),
            pl.BlockSpec((QV, 1), res2),
            pl.BlockSpec((HEADS, HID, CLASSES), res3),
            pl.BlockSpec((1, CLASSES), res2),
        ],
        out_specs=pl.BlockSpec((N, CLASSES), res2),
        out_shape=jax.ShapeDtypeStruct((N, CLASSES), jnp.float32),
        scratch_shapes=[
            pltpu.VMEM((HEADS, HID + 1, N), jnp.bfloat16),
            pltpu.VMEM((HEADS, 1, N), jnp.bfloat16),
            pltpu.VMEM((HEADS, 1, N), jnp.bfloat16),
            pltpu.SMEM((HEADS, 1), jnp.float32),
            pltpu.VMEM((HEADS, N, 3 * HID), jnp.bfloat16),
        ],
    )(x, W, a.reshape(HEADS, 2 * HID, 1), adj0, adj1, adj2,
      Ws.astype(jnp.bfloat16).reshape(HEADS, HID, QV),
      bs.reshape(1, QV), q.reshape(QV, 1),
      Wo.astype(jnp.bfloat16).reshape(HEADS, HID, CLASSES),
      bo.reshape(1, CLASSES))


# speedup vs baseline: 1.0218x; 1.0218x over previous
import jax
import jax.numpy as jnp
from jax.experimental import pallas as pl
from jax.experimental.pallas import tpu as pltpu

N = 2048
FEAT = 128
HID = 32
HEADS = 8
CLASSES = 16
QV = 128
ALPHA = 0.2

BR = 512
NB = N // BR


def _han_kernel(x_ref, W_ref, a_ref, adj0_ref, adj1_ref, adj2_ref,
                Ws_ref, bs_ref, q_ref, Wo_ref, bo_ref, out_ref,
                hbt_s, ex_s, fx_s, f2m_s, z_s):
    i = pl.program_id(0)

    @pl.when(i == 0)
    def _prep():
        for hd in range(HEADS):
            ht = jax.lax.dot_general(
                W_ref[hd], x_ref[...], (((0,), (1,)), ((), ())),
                preferred_element_type=jnp.float32)
            a2 = a_ref[hd, HID:, :]
            f2f = jax.lax.dot_general(
                a2, ht.astype(jnp.bfloat16), (((0,), (0,)), ((), ())),
                preferred_element_type=jnp.float32)
            f2max = jnp.max(f2f)
            f2m_s[hd, 0] = f2max
            ex_s[hd] = jnp.exp(f2f - f2max).astype(jnp.bfloat16)
            fx_s[hd] = jnp.exp(ALPHA * (f2f - f2max)).astype(jnp.bfloat16)
            hbt_s[hd] = jnp.concatenate(
                [ht, jnp.ones((1, N), jnp.float32)],
                axis=0).astype(jnp.bfloat16)

    adjs = (adj0_ref[...], adj1_ref[...], adj2_ref[...])
    for hd in range(HEADS):
        hbt = hbt_s[hd]
        hrows = hbt_s[hd, :HID, pl.ds(i * BR, BR)].astype(jnp.float32)
        f1b_f = jax.lax.dot_general(
            hrows, a_ref[hd, :HID, :], (((0,), (0,)), ((), ())),
            preferred_element_type=jnp.float32)
        mf = f1b_f + f2m_s[hd, 0]
        mb = jnp.where(mf >= 0, mf, ALPHA * mf)
        ci = jnp.exp(mf - mb).astype(jnp.bfloat16)
        di = jnp.exp(ALPHA * mf - mb).astype(jnp.bfloat16)
        pshared = jnp.maximum(ex_s[hd] * ci, fx_s[hd] * di)
        for path in range(3):
            p = pshared * adjs[path].astype(jnp.bfloat16)
            nd = jax.lax.dot_general(
                p, hbt, (((1,), (1,)), ((), ())),
                preferred_element_type=jnp.float32)
            denom = jnp.maximum(nd[:, HID:], 1e-38)
            att = nd[:, :HID] / denom
            z_s[hd, pl.ds(i * BR, BR), HID * path:HID * (path + 1)] = \
                jnp.where(att > 0, att,
                          jnp.exp(att) - 1.0).astype(jnp.bfloat16)

    @pl.when(i == NB - 1)
    def _semantic():
        wbars = []
        ys = []
        for path in range(3):
            t = jnp.zeros((N, QV), dtype=jnp.float32)
            y = jnp.zeros((N, CLASSES), dtype=jnp.float32)
            for h in range(HEADS):
                zh = z_s[h, :, HID * path:HID * (path + 1)]
                t = t + jnp.dot(zh, Ws_ref[h],
                                preferred_element_type=jnp.float32)
                y = y + jnp.dot(zh, Wo_ref[h],
                                preferred_element_type=jnp.float32)
            w = jnp.dot(jnp.tanh(t + bs_ref[...]), q_ref[...],
                        preferred_element_type=jnp.float32)
            wbars.append(jnp.sum(w) / N)
            ys.append(y)
        m = jnp.maximum(jnp.maximum(wbars[0], wbars[1]), wbars[2])
        e0 = jnp.exp(wbars[0] - m)
        e1 = jnp.exp(wbars[1] - m)
        e2 = jnp.exp(wbars[2] - m)
        den = e0 + e1 + e2
        out_ref[...] = (e0 * ys[0] + e1 * ys[1] + e2 * ys[2]) / den \
            + bo_ref[...]


def kernel(x, adj0, adj1, adj2, W, a, Ws, bs, q, Wo, bo):
    adj_spec = pl.BlockSpec((BR, N), lambda i: (i, 0))
    res2 = lambda i: (0, 0)
    res3 = lambda i: (0, 0, 0)
    return pl.pallas_call(
        _han_kernel,
        grid=(NB,),
        in_specs=[
            pl.BlockSpec((N, FEAT), res2),
            pl.BlockSpec((HEADS, FEAT, HID), res3),
            pl.BlockSpec((HEADS, 2 * HID, 1), res3),
            adj_spec, adj_spec, adj_spec,
            pl.BlockSpec((HEADS, HID, QV), res3),
            pl.BlockSpec((1, QV), res2),
            pl.BlockSpec((QV, 1), res2),
            pl.BlockSpec((HEADS, HID, CLASSES), res3),
            pl.BlockSpec((1, CLASSES), res2),
        ],
        out_specs=pl.BlockSpec((N, CLASSES), res2),
        out_shape=jax.ShapeDtypeStruct((N, CLASSES), jnp.float32),
        scratch_shapes=[
            pltpu.VMEM((HEADS, HID + 1, N), jnp.bfloat16),
            pltpu.VMEM((HEADS, 1, N), jnp.bfloat16),
            pltpu.VMEM((HEADS, 1, N), jnp.bfloat16),
            pltpu.SMEM((HEADS, 1), jnp.float32),
            pltpu.VMEM((HEADS, N, 3 * HID), jnp.bfloat16),
        ],
    )(x.astype(jnp.bfloat16), W.astype(jnp.bfloat16),
      a.reshape(HEADS, 2 * HID, 1), adj0, adj1, adj2,
      Ws.astype(jnp.bfloat16).reshape(HEADS, HID, QV),
      bs.reshape(1, QV), q.reshape(QV, 1),
      Wo.astype(jnp.bfloat16).reshape(HEADS, HID, CLASSES),
      bo.reshape(1, CLASSES))


# speedup vs baseline: 1.0462x; 1.0239x over previous
import jax
import jax.numpy as jnp
from jax.experimental import pallas as pl
from jax.experimental.pallas import tpu as pltpu

N = 2048
FEAT = 128
HID = 32
HEADS = 8
CLASSES = 16
QV = 128
ALPHA = 0.2

BR = 512
NB = N // BR


def _han_kernel(x_ref, W_ref, a_ref, adj0_ref, adj1_ref, adj2_ref,
                WsAll_ref, bsAll_ref, qAll_ref, WoAll_ref, bo_ref, out_ref,
                hbt_s, ex_s, fx_s, f2m_s, z_s):
    i = pl.program_id(0)

    @pl.when(i == 0)
    def _prep():
        for hd in range(HEADS):
            ht = jax.lax.dot_general(
                W_ref[hd], x_ref[...], (((0,), (1,)), ((), ())),
                preferred_element_type=jnp.float32)
            a2 = a_ref[hd, HID:, :]
            f2f = jax.lax.dot_general(
                a2, ht, (((0,), (0,)), ((), ())),
                preferred_element_type=jnp.float32)
            f2max = jnp.max(f2f)
            f2m_s[hd, 0] = f2max
            ex_s[hd] = jnp.exp(f2f - f2max).astype(jnp.bfloat16)
            fx_s[hd] = jnp.exp(ALPHA * (f2f - f2max)).astype(jnp.bfloat16)
            hbt_s[hd] = jnp.concatenate(
                [ht, jnp.ones((1, N), jnp.float32)],
                axis=0).astype(jnp.bfloat16)

    adjs = (adj0_ref[...], adj1_ref[...], adj2_ref[...])
    for hd in range(HEADS):
        hbt = hbt_s[hd]
        hrows = hbt_s[hd, :HID, pl.ds(i * BR, BR)].astype(jnp.float32)
        f1b_f = jax.lax.dot_general(
            hrows, a_ref[hd, :HID, :], (((0,), (0,)), ((), ())),
            preferred_element_type=jnp.float32)
        mf = f1b_f + f2m_s[hd, 0]
        mb = jnp.where(mf >= 0, mf, ALPHA * mf)
        ci = jnp.exp(mf - mb).astype(jnp.bfloat16)
        di = jnp.exp(ALPHA * mf - mb).astype(jnp.bfloat16)
        pshared = jnp.maximum(ex_s[hd] * ci, fx_s[hd] * di)
        for path in range(3):
            p = pshared * adjs[path].astype(jnp.bfloat16)
            nd = jax.lax.dot_general(
                p, hbt, (((1,), (1,)), ((), ())),
                preferred_element_type=jnp.float32)
            denom = jnp.maximum(nd[:, HID:], 1e-38)
            att = nd[:, :HID] / denom
            off = 3 * HID * hd + HID * path
            z_s[pl.ds(i * BR, BR), off:off + HID] = \
                jnp.where(att > 0, att,
                          jnp.exp(att) - 1.0).astype(jnp.bfloat16)

    @pl.when(i == NB - 1)
    def _semantic():
        zall = z_s[...]
        tall = jnp.dot(zall, WsAll_ref[...],
                       preferred_element_type=jnp.float32)
        w3 = jnp.dot(jnp.tanh(tall + bsAll_ref[...]), qAll_ref[...],
                     preferred_element_type=jnp.float32)
        wb = jnp.sum(w3, axis=0, keepdims=True) / N
        m = jnp.max(wb)
        e = jnp.exp(wb - m)
        beta = e / jnp.sum(e)
        yall = jnp.dot(zall, WoAll_ref[...],
                       preferred_element_type=jnp.float32)
        out_ref[...] = (beta[0, 0] * yall[:, :CLASSES]
                        + beta[0, 1] * yall[:, CLASSES:2 * CLASSES]
                        + beta[0, 2] * yall[:, 2 * CLASSES:]) + bo_ref[...]


def kernel(x, adj0, adj1, adj2, W, a, Ws, bs, q, Wo, bo):
    adj_spec = pl.BlockSpec((BR, N), lambda i: (i, 0))
    res2 = lambda i: (0, 0)
    res3 = lambda i: (0, 0, 0)
    call = pl.pallas_call(
        _han_kernel,
        grid=(NB,),
        in_specs=[
            pl.BlockSpec((N, FEAT), res2),
            pl.BlockSpec((HEADS, FEAT, HID), res3),
            pl.BlockSpec((HEADS, 2 * HID, 1), res3),
            adj_spec, adj_spec, adj_spec,
            pl.BlockSpec((3 * HEADS * HID, 3 * QV), res2),
            pl.BlockSpec((1, 3 * QV), res2),
            pl.BlockSpec((3 * QV, 3), res2),
            pl.BlockSpec((3 * HEADS * HID, 3 * CLASSES), res2),
            pl.BlockSpec((1, CLASSES), res2),
        ],
        out_specs=pl.BlockSpec((N, CLASSES), res2),
        out_shape=jax.ShapeDtypeStruct((N, CLASSES), jnp.float32),
        scratch_shapes=[
            pltpu.VMEM((HEADS, HID + 1, N), jnp.bfloat16),
            pltpu.VMEM((HEADS, 1, N), jnp.bfloat16),
            pltpu.VMEM((HEADS, 1, N), jnp.bfloat16),
            pltpu.SMEM((HEADS, 1), jnp.float32),
            pltpu.VMEM((N, 3 * HEADS * HID), jnp.bfloat16),
        ],
    )
    eye3 = jnp.eye(3, dtype=jnp.float32)
    WsAll = jnp.einsum('hdq,pr->hpdrq', Ws.reshape(HEADS, HID, QV),
                       eye3).reshape(3 * HEADS * HID, 3 * QV)
    WoAll = jnp.einsum('hdq,pr->hpdrq', Wo.reshape(HEADS, HID, CLASSES),
                       eye3).reshape(3 * HEADS * HID, 3 * CLASSES)
    qAll = jnp.einsum('k,rp->rkp', q, eye3).reshape(3 * QV, 3)
    return call(x, W, a.reshape(HEADS, 2 * HID, 1), adj0, adj1, adj2,
                WsAll.astype(jnp.bfloat16),
                jnp.tile(bs, 3).reshape(1, 3 * QV), qAll,
                WoAll.astype(jnp.bfloat16), bo.reshape(1, CLASSES))


# speedup vs baseline: 1.0503x; 1.0040x over previous
import jax
import jax.numpy as jnp
from jax.experimental import pallas as pl
from jax.experimental.pallas import tpu as pltpu

N = 2048
FEAT = 128
HID = 32
HEADS = 8
CLASSES = 16
QV = 128
ALPHA = 0.2

BR = 512
NB = N // BR


def _han_kernel(x_ref, W_ref, a_ref, adj0_ref, adj1_ref, adj2_ref,
                WsAll_ref, bsAll_ref, qAll_ref, WoAll_ref, bo_ref, out_ref,
                hbt_s, ex_s, fx_s, f2m_s, z_s):
    i = pl.program_id(0)

    @pl.when(i == 0)
    def _prep():
        for hd in range(HEADS):
            ht = jax.lax.dot_general(
                W_ref[hd], x_ref[...], (((0,), (1,)), ((), ())),
                preferred_element_type=jnp.float32)
            a2 = a_ref[hd, HID:, :]
            f2f = jax.lax.dot_general(
                a2, ht, (((0,), (0,)), ((), ())),
                preferred_element_type=jnp.float32)
            f2max = jnp.max(f2f)
            f2m_s[hd, 0] = f2max
            ex_s[hd] = jnp.exp(f2f - f2max).astype(jnp.bfloat16)
            fx_s[hd] = jnp.exp(ALPHA * (f2f - f2max)).astype(jnp.bfloat16)
            hbt_s[hd] = jnp.concatenate(
                [ht, jnp.ones((1, N), jnp.float32)],
                axis=0).astype(jnp.bfloat16)

    adjs = (adj0_ref[...], adj1_ref[...], adj2_ref[...])
    for hd in range(HEADS):
        hbt = hbt_s[hd]
        hrows = hbt_s[hd, :HID, pl.ds(i * BR, BR)].astype(jnp.float32)
        f1b_f = jax.lax.dot_general(
            hrows, a_ref[hd, :HID, :], (((0,), (0,)), ((), ())),
            preferred_element_type=jnp.float32)
        mf = f1b_f + f2m_s[hd, 0]
        mb = jnp.where(mf >= 0, mf, ALPHA * mf)
        ci = jnp.exp(mf - mb).astype(jnp.bfloat16)
        di = jnp.exp(ALPHA * mf - mb).astype(jnp.bfloat16)
        pshared = jnp.maximum(ex_s[hd] * ci, fx_s[hd] * di)
        for path in range(3):
            p = pshared * adjs[path].astype(jnp.bfloat16)
            nd = jax.lax.dot_general(
                p, hbt, (((1,), (1,)), ((), ())),
                preferred_element_type=jnp.float32)
            rden = 1.0 / jnp.maximum(nd[:, HID:], 1e-38)
            att = nd[:, :HID] * rden
            off = 3 * HID * hd + HID * path
            z_s[pl.ds(i * BR, BR), off:off + HID] = \
                jnp.where(att > 0, att,
                          jnp.exp(att) - 1.0).astype(jnp.bfloat16)

    @pl.when(i == NB - 1)
    def _semantic():
        zall = z_s[...]
        tall = jnp.dot(zall, WsAll_ref[...],
                       preferred_element_type=jnp.float32)
        w3 = jnp.dot(jnp.tanh(tall + bsAll_ref[...]), qAll_ref[...],
                     preferred_element_type=jnp.float32)
        wb = jnp.sum(w3, axis=0, keepdims=True) / N
        m = jnp.max(wb)
        e = jnp.exp(wb - m)
        beta = e / jnp.sum(e)
        yall = jnp.dot(zall, WoAll_ref[...],
                       preferred_element_type=jnp.float32)
        out_ref[...] = (beta[0, 0] * yall[:, :CLASSES]
                        + beta[0, 1] * yall[:, CLASSES:2 * CLASSES]
                        + beta[0, 2] * yall[:, 2 * CLASSES:]) + bo_ref[...]


def kernel(x, adj0, adj1, adj2, W, a, Ws, bs, q, Wo, bo):
    adj_spec = pl.BlockSpec((BR, N), lambda i: (i, 0))
    res2 = lambda i: (0, 0)
    res3 = lambda i: (0, 0, 0)
    call = pl.pallas_call(
        _han_kernel,
        grid=(NB,),
        in_specs=[
            pl.BlockSpec((N, FEAT), res2),
            pl.BlockSpec((HEADS, FEAT, HID), res3),
            pl.BlockSpec((HEADS, 2 * HID, 1), res3),
            adj_spec, adj_spec, adj_spec,
            pl.BlockSpec((3 * HEADS * HID, 3 * QV), res2),
            pl.BlockSpec((1, 3 * QV), res2),
            pl.BlockSpec((3 * QV, 3), res2),
            pl.BlockSpec((3 * HEADS * HID, 3 * CLASSES), res2),
            pl.BlockSpec((1, CLASSES), res2),
        ],
        out_specs=pl.BlockSpec((N, CLASSES), res2),
        out_shape=jax.ShapeDtypeStruct((N, CLASSES), jnp.float32),
        scratch_shapes=[
            pltpu.VMEM((HEADS, HID + 1, N), jnp.bfloat16),
            pltpu.VMEM((HEADS, 1, N), jnp.bfloat16),
            pltpu.VMEM((HEADS, 1, N), jnp.bfloat16),
            pltpu.SMEM((HEADS, 1), jnp.float32),
            pltpu.VMEM((N, 3 * HEADS * HID), jnp.bfloat16),
        ],
    )
    eye3 = jnp.eye(3, dtype=jnp.float32)
    WsAll = jnp.einsum('hdq,pr->hpdrq', Ws.reshape(HEADS, HID, QV),
                       eye3).reshape(3 * HEADS * HID, 3 * QV)
    WoAll = jnp.einsum('hdq,pr->hpdrq', Wo.reshape(HEADS, HID, CLASSES),
                       eye3).reshape(3 * HEADS * HID, 3 * CLASSES)
    qAll = jnp.einsum('k,rp->rkp', q, eye3).reshape(3 * QV, 3)
    return call(x, W, a.reshape(HEADS, 2 * HID, 1), adj0, adj1, adj2,
                WsAll.astype(jnp.bfloat16),
                jnp.tile(bs, 3).reshape(1, 3 * QV), qAll,
                WoAll.astype(jnp.bfloat16), bo.reshape(1, CLASSES))
